# baseline (speedup 1.0000x reference)
import jax
import jax.numpy as jnp
import numpy as np
from jax.experimental import pallas as pl
from jax.experimental.pallas import tpu as pltpu

_SEQ = 16384
_D = 512
_BM = 1024
_NB_IN = _SEQ // _BM
_GRID = _NB_IN + 1
_REV = (_SEQ - 1) * 2 + 5
_SB = 128
_NSB = _BM // _SB
_MROWS = _BM // 128
_CROWS = 2 * _BM // 128


def _upsample_body(x_ref, c_ref, m_ref, w_ref, u_ref, v_ref,
                   out_ref, cout_ref, cnt_ref, carry_a, carry_x):
    i = pl.program_id(0)
    lane8 = jax.lax.broadcasted_iota(jnp.int32, (1, 8), 1)

    w = w_ref[...]
    u = u_ref[...]
    v = v_ref[...]
    valid = i < _NB_IN

    prev_a = jnp.where(i == 0, 0.0, carry_a[...])
    prev_x = jnp.where(i == 0, 0.0, carry_x[...])

    asubs = []
    xsubs = []
    for k in range(_NSB):
        rows = slice(k * _SB, (k + 1) * _SB)
        mc = m_ref[k:k + 1, 0:128].reshape(_SB, 1)
        mi = m_ref[k:k + 1, 128:256].reshape(_SB, 1)

        lin = jnp.dot(x_ref[rows, :], w,
                      preferred_element_type=jnp.float32)
        asub = lin * mi
        coord_m = c_ref[rows, :] * mc
        xsub = jnp.concatenate(
            [coord_m, mc, mi, jnp.zeros((_SB, 3), jnp.float32)], axis=1)
        asubs.append(jnp.where(valid, asub, 0.0))
        xsubs.append(jnp.where(valid, xsub, 0.0))

    cnt_cols = []
    for k in range(_NSB):
        pa = prev_a if k == 0 else asubs[k - 1][_SB - 8:_SB, :]
        px = prev_x if k == 0 else xsubs[k - 1][_SB - 8:_SB, :]
        out_raw = (jnp.dot(u, asubs[k], preferred_element_type=jnp.float32)
                   + jnp.dot(v, pa, preferred_element_type=jnp.float32))
        aux_raw = (jnp.dot(u, xsubs[k], preferred_element_type=jnp.float32)
                   + jnp.dot(v, px, preferred_element_type=jnp.float32))

        sl = slice(2 * _SB * k, 2 * _SB * (k + 1))
        out_ref[sl, :] = out_raw / jnp.maximum(aux_raw[:, 4:5], 1.0)
        cout_ref[sl, :] = (aux_raw[:, 0:3]
                           / (jnp.maximum(aux_raw[:, 3:4], 1.0) + 1e-6))
        cnt_cols.append(aux_raw[:, 3:5])

    cnt = jnp.concatenate(cnt_cols, axis=0)
    cnt_ref[:, 0:128] = cnt[:, 0:1].reshape(_CROWS, 128)
    cnt_ref[:, 128:256] = cnt[:, 1:2].reshape(_CROWS, 128)

    carry_a[...] = asubs[-1][_SB - 8:_SB, :]
    carry_x[...] = xsubs[-1][_SB - 8:_SB, :]


def kernel(irreps_array, mask_irreps_array, coord, mask_coord, W):
    mc = mask_coord.astype(jnp.float32).reshape(_SEQ // 128, 128)
    mi = mask_irreps_array.astype(jnp.float32).reshape(_SEQ // 128, 128)
    maskpack = jnp.concatenate([mc, mi], axis=1)

    r_idx = np.arange(2 * _SB)[:, None]
    c_idx = np.arange(_SB)[None, :]
    t = r_idx - 2 * c_idx
    u_mat = jnp.asarray(((t >= 0) & (t <= 4)).astype(np.float32))
    c8 = np.arange(8)[None, :]
    tv = r_idx + 16 - 2 * c8
    v_mat = jnp.asarray(((tv >= 0) & (tv <= 4)).astype(np.float32))

    n_cnt_rows = _GRID * _CROWS
    out, cout, cntpack = pl.pallas_call(
        _upsample_body,
        grid=(_GRID,),
        in_specs=[
            pl.BlockSpec((_BM, _D), lambda i: (jnp.minimum(i, _NB_IN - 1), 0)),
            pl.BlockSpec((_BM, 3), lambda i: (jnp.minimum(i, _NB_IN - 1), 0)),
            pl.BlockSpec((_MROWS, 256),
                         lambda i: (jnp.minimum(i, _NB_IN - 1), 0)),
            pl.BlockSpec((_D, _D), lambda i: (0, 0)),
            pl.BlockSpec((2 * _SB, _SB), lambda i: (0, 0)),
            pl.BlockSpec((2 * _SB, 8), lambda i: (0, 0)),
        ],
        out_specs=[
            pl.BlockSpec((2 * _BM, _D), lambda i: (i, 0)),
            pl.BlockSpec((2 * _BM, 3), lambda i: (i, 0)),
            pl.BlockSpec((_CROWS, 256), lambda i: (i, 0)),
        ],
        out_shape=[
            jax.ShapeDtypeStruct((_REV, _D), jnp.float32),
            jax.ShapeDtypeStruct((_REV, 3), jnp.float32),
            jax.ShapeDtypeStruct((n_cnt_rows, 256), jnp.float32),
        ],
        scratch_shapes=[
            pltpu.VMEM((8, _D), jnp.float32),
            pltpu.VMEM((8, 8), jnp.float32),
        ],
        compiler_params=pltpu.CompilerParams(
            dimension_semantics=("arbitrary",)),
    )(irreps_array, coord, maskpack, W, u_mat, v_mat)

    cnt_c = cntpack[:, 0:128].reshape(-1)[:_REV]
    cnt_i = cntpack[:, 128:256].reshape(-1)[:_REV]
    return out, cnt_i > 0.0, cout, cnt_c > 0.0


# speedup vs baseline: 1.0629x; 1.0629x over previous
import jax
import jax.numpy as jnp
import numpy as np
from jax.experimental import pallas as pl
from jax.experimental.pallas import tpu as pltpu

_SEQ = 16384
_D = 512
_BM = 2048
_NB_IN = _SEQ // _BM
_GRID = _NB_IN + 1
_REV = (_SEQ - 1) * 2 + 5
_SB = 128
_NSB = _BM // _SB
_MROWS = _BM // 128
_CROWS = 2 * _BM // 128


def _upsample_body(x_ref, c_ref, m_ref, w_ref, u_ref, v_ref,
                   out_ref, cout_ref, cnt_ref, carry_a, carry_x):
    i = pl.program_id(0)
    lane8 = jax.lax.broadcasted_iota(jnp.int32, (1, 8), 1)

    w = w_ref[...]
    u = u_ref[...]
    v = v_ref[...]
    valid = i < _NB_IN

    prev_a = jnp.where(i == 0, 0.0, carry_a[...])
    prev_x = jnp.where(i == 0, 0.0, carry_x[...])

    asubs = []
    xsubs = []
    for k in range(_NSB):
        rows = slice(k * _SB, (k + 1) * _SB)
        mc = m_ref[k:k + 1, 0:128].reshape(_SB, 1)
        mi = m_ref[k:k + 1, 128:256].reshape(_SB, 1)

        lin = jnp.dot(x_ref[rows, :], w,
                      preferred_element_type=jnp.float32)
        asub = lin * mi
        coord_m = c_ref[rows, :] * mc
        xsub = jnp.concatenate(
            [coord_m, mc, mi, jnp.zeros((_SB, 3), jnp.float32)], axis=1)
        asubs.append(jnp.where(valid, asub, 0.0))
        xsubs.append(jnp.where(valid, xsub, 0.0))

    cnt_cols = []
    for k in range(_NSB):
        pa = prev_a if k == 0 else asubs[k - 1][_SB - 8:_SB, :]
        px = prev_x if k == 0 else xsubs[k - 1][_SB - 8:_SB, :]
        out_raw = (jnp.dot(u, asubs[k], preferred_element_type=jnp.float32)
                   + jnp.dot(v, pa, preferred_element_type=jnp.float32))
        aux_raw = (jnp.dot(u, xsubs[k], preferred_element_type=jnp.float32)
                   + jnp.dot(v, px, preferred_element_type=jnp.float32))

        sl = slice(2 * _SB * k, 2 * _SB * (k + 1))
        out_ref[sl, :] = out_raw / jnp.maximum(aux_raw[:, 4:5], 1.0)
        cout_ref[sl, :] = (aux_raw[:, 0:3]
                           / (jnp.maximum(aux_raw[:, 3:4], 1.0) + 1e-6))
        cnt_cols.append(aux_raw[:, 3:5])

    cnt = jnp.concatenate(cnt_cols, axis=0)
    cnt_ref[:, 0:128] = cnt[:, 0:1].reshape(_CROWS, 128)
    cnt_ref[:, 128:256] = cnt[:, 1:2].reshape(_CROWS, 128)

    carry_a[...] = asubs[-1][_SB - 8:_SB, :]
    carry_x[...] = xsubs[-1][_SB - 8:_SB, :]


def kernel(irreps_array, mask_irreps_array, coord, mask_coord, W):
    mc = mask_coord.astype(jnp.float32).reshape(_SEQ // 128, 128)
    mi = mask_irreps_array.astype(jnp.float32).reshape(_SEQ // 128, 128)
    maskpack = jnp.concatenate([mc, mi], axis=1)

    r_idx = np.arange(2 * _SB)[:, None]
    c_idx = np.arange(_SB)[None, :]
    t = r_idx - 2 * c_idx
    u_mat = jnp.asarray(((t >= 0) & (t <= 4)).astype(np.float32))
    c8 = np.arange(8)[None, :]
    tv = r_idx + 16 - 2 * c8
    v_mat = jnp.asarray(((tv >= 0) & (tv <= 4)).astype(np.float32))

    n_cnt_rows = _GRID * _CROWS
    out, cout, cntpack = pl.pallas_call(
        _upsample_body,
        grid=(_GRID,),
        in_specs=[
            pl.BlockSpec((_BM, _D), lambda i: (jnp.minimum(i, _NB_IN - 1), 0)),
            pl.BlockSpec((_BM, 3), lambda i: (jnp.minimum(i, _NB_IN - 1), 0)),
            pl.BlockSpec((_MROWS, 256),
                         lambda i: (jnp.minimum(i, _NB_IN - 1), 0)),
            pl.BlockSpec((_D, _D), lambda i: (0, 0)),
            pl.BlockSpec((2 * _SB, _SB), lambda i: (0, 0)),
            pl.BlockSpec((2 * _SB, 8), lambda i: (0, 0)),
        ],
        out_specs=[
            pl.BlockSpec((2 * _BM, _D), lambda i: (i, 0)),
            pl.BlockSpec((2 * _BM, 3), lambda i: (i, 0)),
            pl.BlockSpec((_CROWS, 256), lambda i: (i, 0)),
        ],
        out_shape=[
            jax.ShapeDtypeStruct((_REV, _D), jnp.float32),
            jax.ShapeDtypeStruct((_REV, 3), jnp.float32),
            jax.ShapeDtypeStruct((n_cnt_rows, 256), jnp.float32),
        ],
        scratch_shapes=[
            pltpu.VMEM((8, _D), jnp.float32),
            pltpu.VMEM((8, 8), jnp.float32),
        ],
        compiler_params=pltpu.CompilerParams(
            dimension_semantics=("arbitrary",)),
    )(irreps_array, coord, maskpack, W, u_mat, v_mat)

    cnt_c = cntpack[:, 0:128].reshape(-1)[:_REV]
    cnt_i = cntpack[:, 128:256].reshape(-1)[:_REV]
    return out, cnt_i > 0.0, cout, cnt_c > 0.0
